# Initial kernel scaffold; baseline (speedup 1.0000x reference)
import jax
import jax.numpy as jnp
from jax.experimental import pallas as pl


def kernel(action, table):
    raise NotImplementedError("write your pallas kernel here")


# speedup vs baseline: 1.9466x; 1.9466x over previous
import functools

import jax
import jax.numpy as jnp
from jax import lax
from jax.experimental import pallas as pl
from jax.experimental.pallas import tpu as pltpu
from jax.experimental.pallas import tpu_sc as plsc

_B = 16384
_HIST = 20
_D = 32
_TOTAL = _B * _HIST
_NC = 2
_NS = 16
_NW = _NC * _NS
_PER_W = _TOTAL // _NW
_CHUNK = 512
_NBUF = 4
_N_CHUNKS = _PER_W // _CHUNK
_SLACK = 2


def _sc_gather(idx_flat, table):
    mesh = plsc.VectorSubcoreMesh(core_axis_name="c", subcore_axis_name="s")

    @functools.partial(
        pl.kernel,
        mesh=mesh,
        out_type=jax.ShapeDtypeStruct((_TOTAL, _D), jnp.float32),
        scratch_types=[
            pltpu.VMEM((_PER_W,), jnp.int32),
            [pltpu.VMEM((_CHUNK, _D), jnp.float32) for _ in range(_NBUF)],
            [pltpu.SemaphoreType.DMA for _ in range(_NBUF)],
            [pltpu.SemaphoreType.DMA for _ in range(_NBUF)],
        ],
        compiler_params=pltpu.CompilerParams(use_tc_tiling_on_sc=False),
    )
    def k(idx_hbm, table_hbm, out_hbm, idx_v, rows, gsem, wsem):
        wid = lax.axis_index("s") * _NC + lax.axis_index("c")
        base = wid * _PER_W
        pltpu.sync_copy(idx_hbm.at[pl.ds(base, _PER_W)], idx_v)

        gcopies = [None] * _NBUF
        wcopies = [None] * _NBUF

        def start_gather(i):
            b = i % _NBUF
            if wcopies[b] is not None:
                wcopies[b].wait()
            gcopies[b] = pltpu.async_copy(
                table_hbm.at[idx_v.at[pl.ds(i * _CHUNK, _CHUNK)]],
                rows[b], gsem[b])

        def start_write(i):
            b = i % _NBUF
            gcopies[b].wait()
            wcopies[b] = pltpu.async_copy(
                rows[b], out_hbm.at[pl.ds(base + i * _CHUNK, _CHUNK)],
                wsem[b])

        for i in range(_N_CHUNKS + _SLACK):
            if i < _N_CHUNKS:
                start_gather(i)
            if i >= _SLACK:
                start_write(i - _SLACK)
        for i in range(_NBUF):
            b = (_N_CHUNKS - 1 - i) % _NBUF
            if wcopies[b] is not None:
                wcopies[b].wait()

    return k(idx_flat, table)


def kernel(action, table):
    idx = action.reshape(-1).astype(jnp.int32)
    out = _sc_gather(idx, table)
    return out.reshape(_B, _HIST * _D)
